# baseline (speedup 1.0000x reference)
import functools

import jax
import jax.numpy as jnp
from jax import lax
from jax.experimental import pallas as pl
from jax.experimental.pallas import tpu as pltpu
from jax.experimental.pallas import tpu_sc as plsc

_NC = 2
_NS = 16
_NW = _NC * _NS


def _sc_gather_pair(xl, xr, src, dst):
    N, D = xl.shape
    E = src.shape[0]
    per_w = E // _NW
    C = next(c for c in (80, 40, 16, 8) if per_w % c == 0)
    steps = per_w // C
    main_steps = steps - (steps % 2)

    mesh = plsc.VectorSubcoreMesh(core_axis_name="c", subcore_axis_name="s")

    @functools.partial(
        pl.kernel,
        mesh=mesh,
        out_type=[
            jax.ShapeDtypeStruct((E, D), jnp.float32),
            jax.ShapeDtypeStruct((E, D), jnp.float32),
        ],
        scratch_types=[
            pltpu.VMEM((per_w,), jnp.int32),
            pltpu.VMEM((2, C, D), jnp.float32),
            pltpu.SemaphoreType.DMA((2,)),
            pltpu.SemaphoreType.DMA((2,)),
            pltpu.SemaphoreType.DMA,
        ],
    )
    def k(xl_hbm, xr_hbm, src_hbm, dst_hbm, gl_hbm, gr_hbm,
          idx_v, rows_v, gsem, osem, isem):
        wid = lax.axis_index("s") * _NC + lax.axis_index("c")
        base = wid * per_w

        def one_table(table_hbm, idx_hbm, out_hbm):
            pltpu.async_copy(idx_hbm.at[pl.ds(base, per_w)], idx_v, isem).wait()

            def gather(g, b):
                return pltpu.make_async_copy(
                    table_hbm.at[idx_v.at[pl.ds(g * C, C)]],
                    rows_v.at[b], gsem.at[b])

            def out_copy(g, b):
                return pltpu.make_async_copy(
                    rows_v.at[b], out_hbm.at[pl.ds(base + g * C, C)],
                    osem.at[b])

            gather(0, 0).start()

            @pl.loop(0, main_steps, step=2)
            def _(g0):
                for b in range(2):
                    g = g0 + b
                    nb = 1 - b
                    gather(g, b).wait()

                    @pl.when(g + 1 < steps)
                    def _():
                        @pl.when(g >= 1)
                        def _():
                            out_copy(g - 1, nb).wait()
                        gather(g + 1, nb).start()

                    out_copy(g, b).start()

            if steps % 2 == 1:
                gather(steps - 1, (steps - 1) % 2).wait()
                out_copy(steps - 1, (steps - 1) % 2).start()
            out_copy(steps - 2, (steps - 2) % 2).wait()
            out_copy(steps - 1, (steps - 1) % 2).wait()

        one_table(xl_hbm, src_hbm, gl_hbm)
        one_table(xr_hbm, dst_hbm, gr_hbm)

    return k(xl, xr, src, dst)


def _enc_body(x_ref, Wn_ref, bn_ref, Wl_ref, bl_ref, Wr_ref, br_ref,
              xl_ref, xr_ref):
    h = jnp.maximum(x_ref[...] @ Wn_ref[...] + bn_ref[...], 0.0)
    xl_ref[...] = h @ Wl_ref[...] + bl_ref[...]
    xr_ref[...] = h @ Wr_ref[...] + br_ref[...]


def _node_encode(x, Wn, bn, Wl, bl, Wr, br):
    N, F = x.shape
    HH = Wl.shape[1]
    BN = 2000
    grid = (N // BN,)
    return pl.pallas_call(
        _enc_body,
        grid=grid,
        in_specs=[
            pl.BlockSpec((BN, F), lambda i: (i, 0)),
            pl.BlockSpec(Wn.shape, lambda i: (0, 0)),
            pl.BlockSpec((1, bn.shape[1]), lambda i: (0, 0)),
            pl.BlockSpec(Wl.shape, lambda i: (0, 0)),
            pl.BlockSpec((1, bl.shape[1]), lambda i: (0, 0)),
            pl.BlockSpec(Wr.shape, lambda i: (0, 0)),
            pl.BlockSpec((1, br.shape[1]), lambda i: (0, 0)),
        ],
        out_specs=[
            pl.BlockSpec((BN, HH), lambda i: (i, 0)),
            pl.BlockSpec((BN, HH), lambda i: (i, 0)),
        ],
        out_shape=[
            jax.ShapeDtypeStruct((N, HH), jnp.float32),
            jax.ShapeDtypeStruct((N, HH), jnp.float32),
        ],
    )(x, Wn, bn, Wl, bl, Wr, br)


def _logit_body(gl_ref, gr_ref, ea_ref, We_ref, be_ref, Wed_ref, attbd_ref,
                expl_ref, w_ref):
    ea = jnp.maximum(ea_ref[...] @ We_ref[...] + be_ref[...], 0.0)
    gl = gl_ref[...]
    m = gl + gr_ref[...] + ea @ Wed_ref[...]
    s = jnp.where(m >= 0, m, 0.2 * m)
    expl = jnp.exp(s @ attbd_ref[...])
    expl_ref[...] = expl
    H = expl.shape[1]
    HID = gl.shape[1] // H
    w_ref[...] = jnp.concatenate(
        [expl[:, h:h + 1] * gl[:, h * HID:(h + 1) * HID] for h in range(H)],
        axis=1)


def _edge_weights(gl, gr, edge_attr, We, be, Wed, att_bd):
    E, HH = gl.shape
    FE = edge_attr.shape[1]
    H = att_bd.shape[1]
    BE = 2000
    grid = (E // BE,)
    return pl.pallas_call(
        _logit_body,
        grid=grid,
        in_specs=[
            pl.BlockSpec((BE, HH), lambda i: (i, 0)),
            pl.BlockSpec((BE, HH), lambda i: (i, 0)),
            pl.BlockSpec((BE, FE), lambda i: (i, 0)),
            pl.BlockSpec(We.shape, lambda i: (0, 0)),
            pl.BlockSpec((1, be.shape[1]), lambda i: (0, 0)),
            pl.BlockSpec(Wed.shape, lambda i: (0, 0)),
            pl.BlockSpec(att_bd.shape, lambda i: (0, 0)),
        ],
        out_specs=[
            pl.BlockSpec((BE, H), lambda i: (i, 0)),
            pl.BlockSpec((BE, HH), lambda i: (i, 0)),
        ],
        out_shape=[
            jax.ShapeDtypeStruct((E, H), jnp.float32),
            jax.ShapeDtypeStruct((E, HH), jnp.float32),
        ],
    )(gl, gr, edge_attr, We, be, Wed, att_bd)


def _dec_body(aggw_ref, den_ref, bc_ref, Wd1_ref, bd1_ref, Wd2_ref, bd2_ref,
              out_ref):
    aggw = aggw_ref[...]
    den = den_ref[...]
    H = den.shape[1]
    HID = aggw.shape[1] // H
    c = jnp.concatenate(
        [aggw[:, h * HID:(h + 1) * HID] / (den[:, h:h + 1] + 1e-16)
         for h in range(H)], axis=1) + bc_ref[...]
    d = jnp.maximum(c @ Wd1_ref[...] + bd1_ref[...], 0.0)
    out_ref[...] = d @ Wd2_ref[...] + bd2_ref[...]


def _decode(aggw, den, bias_conv, Wd1, bd1, Wd2, bd2):
    N, HH = aggw.shape
    OUT = Wd2.shape[1]
    BN = 2000
    grid = (N // BN,)
    return pl.pallas_call(
        _dec_body,
        grid=grid,
        in_specs=[
            pl.BlockSpec((BN, HH), lambda i: (i, 0)),
            pl.BlockSpec((BN, den.shape[1]), lambda i: (i, 0)),
            pl.BlockSpec((1, HH), lambda i: (0, 0)),
            pl.BlockSpec(Wd1.shape, lambda i: (0, 0)),
            pl.BlockSpec((1, bd1.shape[1]), lambda i: (0, 0)),
            pl.BlockSpec(Wd2.shape, lambda i: (0, 0)),
            pl.BlockSpec((1, bd2.shape[1]), lambda i: (0, 0)),
        ],
        out_specs=pl.BlockSpec((BN, OUT), lambda i: (i, 0)),
        out_shape=jax.ShapeDtypeStruct((N, OUT), jnp.float32),
    )(aggw, den, bias_conv, Wd1, bd1, Wd2, bd2)


def kernel(x, edge_index, edge_attr, Wn, bn, We, be, Wl, bl, Wr, br, Wed,
           att, bias_conv, Wd1, bd1, Wd2, bd2):
    N = x.shape[0]
    E = edge_index.shape[1]
    HEADS, HID = att.shape
    HH = HEADS * HID

    src = edge_index[0]
    dst = edge_index[1]

    att_bd = jnp.zeros((HH, HEADS), jnp.float32)
    for h in range(HEADS):
        att_bd = att_bd.at[h * HID:(h + 1) * HID, h].set(att[h])

    xl, xr = _node_encode(x, Wn, bn.reshape(1, -1), Wl, bl.reshape(1, -1),
                          Wr, br.reshape(1, -1))

    K = 2
    EC = E // K
    den = jnp.zeros((N, HEADS), jnp.float32)
    aggw = jnp.zeros((N, HH), jnp.float32)
    for k in range(K):
        sl = slice(k * EC, (k + 1) * EC)
        src_k, dst_k = src[sl], dst[sl]
        gl, gr = _sc_gather_pair(xl, xr, src_k, dst_k)
        expl, w = _edge_weights(gl, gr, edge_attr[sl], We,
                                be.reshape(1, -1), Wed, att_bd)
        den = den + jax.ops.segment_sum(expl, dst_k, num_segments=N)
        aggw = aggw + jax.ops.segment_sum(w, dst_k, num_segments=N)


    return _decode(aggw, den, bias_conv.reshape(1, -1), Wd1,
                   bd1.reshape(1, -1), Wd2, bd2.reshape(1, -1))


# speedup vs baseline: 1.1720x; 1.1720x over previous
import functools

import jax
import jax.numpy as jnp
from jax import lax
from jax.experimental import pallas as pl
from jax.experimental.pallas import tpu as pltpu
from jax.experimental.pallas import tpu_sc as plsc

_NC = 2
_NS = 16
_NW = _NC * _NS


def _sc_gather_pair(xl, xr, src, dst):
    N, D = xl.shape
    E = src.shape[0]
    per_w = E // _NW
    C = next(c for c in (80, 40, 16, 8) if per_w % c == 0)
    steps = per_w // C
    main_steps = steps - (steps % 2)

    mesh = plsc.VectorSubcoreMesh(core_axis_name="c", subcore_axis_name="s")

    @functools.partial(
        pl.kernel,
        mesh=mesh,
        out_type=[
            jax.ShapeDtypeStruct((E, D), jnp.float32),
            jax.ShapeDtypeStruct((E, D), jnp.float32),
        ],
        scratch_types=[
            pltpu.VMEM((per_w,), jnp.int32),
            pltpu.VMEM((2, C, D), jnp.float32),
            pltpu.SemaphoreType.DMA((2,)),
            pltpu.SemaphoreType.DMA((2,)),
            pltpu.SemaphoreType.DMA,
        ],
    )
    def k(xl_hbm, xr_hbm, src_hbm, dst_hbm, gl_hbm, gr_hbm,
          idx_v, rows_v, gsem, osem, isem):
        wid = lax.axis_index("s") * _NC + lax.axis_index("c")
        base = wid * per_w

        def one_table(table_hbm, idx_hbm, out_hbm):
            pltpu.async_copy(idx_hbm.at[pl.ds(base, per_w)], idx_v, isem).wait()

            def gather(g, b):
                return pltpu.make_async_copy(
                    table_hbm.at[idx_v.at[pl.ds(g * C, C)]],
                    rows_v.at[b], gsem.at[b])

            def out_copy(g, b):
                return pltpu.make_async_copy(
                    rows_v.at[b], out_hbm.at[pl.ds(base + g * C, C)],
                    osem.at[b])

            gather(0, 0).start()

            @pl.loop(0, main_steps, step=2)
            def _(g0):
                for b in range(2):
                    g = g0 + b
                    nb = 1 - b
                    gather(g, b).wait()

                    @pl.when(g + 1 < steps)
                    def _():
                        @pl.when(g >= 1)
                        def _():
                            out_copy(g - 1, nb).wait()
                        gather(g + 1, nb).start()

                    out_copy(g, b).start()

            if steps % 2 == 1:
                gather(steps - 1, (steps - 1) % 2).wait()
                out_copy(steps - 1, (steps - 1) % 2).start()
            out_copy(steps - 2, (steps - 2) % 2).wait()
            out_copy(steps - 1, (steps - 1) % 2).wait()

        one_table(xl_hbm, src_hbm, gl_hbm)
        one_table(xr_hbm, dst_hbm, gr_hbm)

    return k(xl, xr, src, dst)


def _enc_body(x_ref, Wn_ref, bn_ref, Wl_ref, bl_ref, Wr_ref, br_ref,
              xl_ref, xr_ref):
    h = jnp.maximum(x_ref[...] @ Wn_ref[...] + bn_ref[...], 0.0)
    xl_ref[...] = h @ Wl_ref[...] + bl_ref[...]
    xr_ref[...] = h @ Wr_ref[...] + br_ref[...]


def _node_encode(x, Wn, bn, Wl, bl, Wr, br):
    N, F = x.shape
    HH = Wl.shape[1]
    BN = 2000
    grid = (N // BN,)
    return pl.pallas_call(
        _enc_body,
        grid=grid,
        in_specs=[
            pl.BlockSpec((BN, F), lambda i: (i, 0)),
            pl.BlockSpec(Wn.shape, lambda i: (0, 0)),
            pl.BlockSpec((1, bn.shape[1]), lambda i: (0, 0)),
            pl.BlockSpec(Wl.shape, lambda i: (0, 0)),
            pl.BlockSpec((1, bl.shape[1]), lambda i: (0, 0)),
            pl.BlockSpec(Wr.shape, lambda i: (0, 0)),
            pl.BlockSpec((1, br.shape[1]), lambda i: (0, 0)),
        ],
        out_specs=[
            pl.BlockSpec((BN, HH), lambda i: (i, 0)),
            pl.BlockSpec((BN, HH), lambda i: (i, 0)),
        ],
        out_shape=[
            jax.ShapeDtypeStruct((N, HH), jnp.float32),
            jax.ShapeDtypeStruct((N, HH), jnp.float32),
        ],
    )(x, Wn, bn, Wl, bl, Wr, br)


_WPAD = 12


def _logit_body(gl_ref, gr_ref, ea_ref, We_ref, be_ref, Wed_ref, attbd_ref,
                w_ref):
    ea = jnp.maximum(ea_ref[...] @ We_ref[...] + be_ref[...], 0.0)
    gl = gl_ref[...]
    m = gl + gr_ref[...] + ea @ Wed_ref[...]
    s = jnp.where(m >= 0, m, 0.2 * m)
    expl = jnp.exp(s @ attbd_ref[...])
    H = expl.shape[1]
    HID = gl.shape[1] // H
    w_ref[...] = jnp.concatenate(
        [expl[:, h:h + 1] * gl[:, h * HID:(h + 1) * HID] for h in range(H)]
        + [expl, jnp.zeros((gl.shape[0], _WPAD), jnp.float32)],
        axis=1)


def _edge_weights(gl, gr, edge_attr, We, be, Wed, att_bd):
    E, HH = gl.shape
    FE = edge_attr.shape[1]
    H = att_bd.shape[1]
    BE = 2000
    grid = (E // BE,)
    return pl.pallas_call(
        _logit_body,
        grid=grid,
        in_specs=[
            pl.BlockSpec((BE, HH), lambda i: (i, 0)),
            pl.BlockSpec((BE, HH), lambda i: (i, 0)),
            pl.BlockSpec((BE, FE), lambda i: (i, 0)),
            pl.BlockSpec(We.shape, lambda i: (0, 0)),
            pl.BlockSpec((1, be.shape[1]), lambda i: (0, 0)),
            pl.BlockSpec(Wed.shape, lambda i: (0, 0)),
            pl.BlockSpec(att_bd.shape, lambda i: (0, 0)),
        ],
        out_specs=pl.BlockSpec((BE, HH + H + _WPAD), lambda i: (i, 0)),
        out_shape=jax.ShapeDtypeStruct((E, HH + H + _WPAD), jnp.float32),
    )(gl, gr, edge_attr, We, be, Wed, att_bd)


def _dec_body(aggw_ref, bc_ref, Wd1_ref, bd1_ref, Wd2_ref, bd2_ref, out_ref):
    aggw = aggw_ref[...]
    HH = Wd1_ref.shape[0]
    H = aggw.shape[1] - HH - _WPAD
    HID = HH // H
    den = aggw[:, HH:HH + H]
    c = jnp.concatenate(
        [aggw[:, h * HID:(h + 1) * HID] / (den[:, h:h + 1] + 1e-16)
         for h in range(H)], axis=1) + bc_ref[...]
    d = jnp.maximum(c @ Wd1_ref[...] + bd1_ref[...], 0.0)
    out_ref[...] = d @ Wd2_ref[...] + bd2_ref[...]


def _decode(aggw, bias_conv, Wd1, bd1, Wd2, bd2):
    N, W = aggw.shape
    HH = Wd1.shape[0]
    OUT = Wd2.shape[1]
    BN = 2000
    grid = (N // BN,)
    return pl.pallas_call(
        _dec_body,
        grid=grid,
        in_specs=[
            pl.BlockSpec((BN, W), lambda i: (i, 0)),
            pl.BlockSpec((1, HH), lambda i: (0, 0)),
            pl.BlockSpec(Wd1.shape, lambda i: (0, 0)),
            pl.BlockSpec((1, bd1.shape[1]), lambda i: (0, 0)),
            pl.BlockSpec(Wd2.shape, lambda i: (0, 0)),
            pl.BlockSpec((1, bd2.shape[1]), lambda i: (0, 0)),
        ],
        out_specs=pl.BlockSpec((BN, OUT), lambda i: (i, 0)),
        out_shape=jax.ShapeDtypeStruct((N, OUT), jnp.float32),
    )(aggw, bias_conv, Wd1, bd1, Wd2, bd2)


def kernel(x, edge_index, edge_attr, Wn, bn, We, be, Wl, bl, Wr, br, Wed,
           att, bias_conv, Wd1, bd1, Wd2, bd2):
    N = x.shape[0]
    E = edge_index.shape[1]
    HEADS, HID = att.shape
    HH = HEADS * HID

    src = edge_index[0]
    dst = edge_index[1]

    att_bd = jnp.zeros((HH, HEADS), jnp.float32)
    for h in range(HEADS):
        att_bd = att_bd.at[h * HID:(h + 1) * HID, h].set(att[h])

    xl, xr = _node_encode(x, Wn, bn.reshape(1, -1), Wl, bl.reshape(1, -1),
                          Wr, br.reshape(1, -1))

    gl, gr = _sc_gather_pair(xl, xr, src, dst)

    w = _edge_weights(gl, gr, edge_attr, We, be.reshape(1, -1), Wed, att_bd)

    aggw = jax.ops.segment_sum(w, dst, num_segments=N)


    return _decode(aggw, bias_conv.reshape(1, -1), Wd1,
                   bd1.reshape(1, -1), Wd2, bd2.reshape(1, -1))


# speedup vs baseline: 1.3623x; 1.1623x over previous
import functools

import jax
import jax.numpy as jnp
from jax import lax
from jax.experimental import pallas as pl
from jax.experimental.pallas import tpu as pltpu
from jax.experimental.pallas import tpu_sc as plsc

_NC = 2
_NS = 16
_NW = _NC * _NS


def _sc_gather_pair(xl, xr, src, dst):
    N, D = xl.shape
    E = src.shape[0]
    per_w = E // _NW
    C = next(c for c in (80, 40, 16, 8) if per_w % c == 0)
    steps = per_w // C
    main_steps = steps - (steps % 2)

    mesh = plsc.VectorSubcoreMesh(core_axis_name="c", subcore_axis_name="s")

    @functools.partial(
        pl.kernel,
        mesh=mesh,
        out_type=[
            jax.ShapeDtypeStruct((E, D), xl.dtype),
            jax.ShapeDtypeStruct((E, D), xr.dtype),
        ],
        scratch_types=[
            pltpu.VMEM((per_w,), jnp.int32),
            pltpu.VMEM((2, C, D), xl.dtype),
            pltpu.SemaphoreType.DMA((2,)),
            pltpu.SemaphoreType.DMA((2,)),
            pltpu.SemaphoreType.DMA,
        ],
    )
    def k(xl_hbm, xr_hbm, src_hbm, dst_hbm, gl_hbm, gr_hbm,
          idx_v, rows_v, gsem, osem, isem):
        wid = lax.axis_index("s") * _NC + lax.axis_index("c")
        base = wid * per_w

        def one_table(table_hbm, idx_hbm, out_hbm):
            pltpu.async_copy(idx_hbm.at[pl.ds(base, per_w)], idx_v, isem).wait()

            def gather(g, b):
                return pltpu.make_async_copy(
                    table_hbm.at[idx_v.at[pl.ds(g * C, C)]],
                    rows_v.at[b], gsem.at[b])

            def out_copy(g, b):
                return pltpu.make_async_copy(
                    rows_v.at[b], out_hbm.at[pl.ds(base + g * C, C)],
                    osem.at[b])

            gather(0, 0).start()

            @pl.loop(0, main_steps, step=2)
            def _(g0):
                for b in range(2):
                    g = g0 + b
                    nb = 1 - b
                    gather(g, b).wait()

                    @pl.when(g + 1 < steps)
                    def _():
                        @pl.when(g >= 1)
                        def _():
                            out_copy(g - 1, nb).wait()
                        gather(g + 1, nb).start()

                    out_copy(g, b).start()

            if steps % 2 == 1:
                gather(steps - 1, (steps - 1) % 2).wait()
                out_copy(steps - 1, (steps - 1) % 2).start()
            out_copy(steps - 2, (steps - 2) % 2).wait()
            out_copy(steps - 1, (steps - 1) % 2).wait()

        one_table(xl_hbm, src_hbm, gl_hbm)
        one_table(xr_hbm, dst_hbm, gr_hbm)

    return k(xl, xr, src, dst)


def _enc_body(x_ref, Wn_ref, bn_ref, Wl_ref, bl_ref, Wr_ref, br_ref,
              xl_ref, xr_ref):
    h = jnp.maximum(x_ref[...] @ Wn_ref[...] + bn_ref[...], 0.0)

    def pack(v):
        half = v.shape[1] // 2
        lo = v[:, :half].astype(jnp.bfloat16).astype(jnp.float32)
        hi = v[:, half:].astype(jnp.bfloat16).astype(jnp.float32)
        lo_u = jax.lax.bitcast_convert_type(lo, jnp.uint32) >> 16
        hi_u = jax.lax.bitcast_convert_type(hi, jnp.uint32) & jnp.uint32(
            0xFFFF0000)
        return lo_u | hi_u

    xl_ref[...] = pack(h @ Wl_ref[...] + bl_ref[...])
    xr_ref[...] = pack(h @ Wr_ref[...] + br_ref[...])


def _node_encode(x, Wn, bn, Wl, bl, Wr, br):
    N, F = x.shape
    HH = Wl.shape[1]
    BN = 2000
    grid = (N // BN,)
    return pl.pallas_call(
        _enc_body,
        grid=grid,
        in_specs=[
            pl.BlockSpec((BN, F), lambda i: (i, 0)),
            pl.BlockSpec(Wn.shape, lambda i: (0, 0)),
            pl.BlockSpec((1, bn.shape[1]), lambda i: (0, 0)),
            pl.BlockSpec(Wl.shape, lambda i: (0, 0)),
            pl.BlockSpec((1, bl.shape[1]), lambda i: (0, 0)),
            pl.BlockSpec(Wr.shape, lambda i: (0, 0)),
            pl.BlockSpec((1, br.shape[1]), lambda i: (0, 0)),
        ],
        out_specs=[
            pl.BlockSpec((BN, HH // 2), lambda i: (i, 0)),
            pl.BlockSpec((BN, HH // 2), lambda i: (i, 0)),
        ],
        out_shape=[
            jax.ShapeDtypeStruct((N, HH // 2), jnp.uint32),
            jax.ShapeDtypeStruct((N, HH // 2), jnp.uint32),
        ],
    )(x, Wn, bn, Wl, bl, Wr, br)


_WPAD = 12


def _unpack_bf16_pair(p):
    lo = jax.lax.bitcast_convert_type(p << 16, jnp.float32)
    hi = jax.lax.bitcast_convert_type(p & jnp.uint32(0xFFFF0000), jnp.float32)
    return lo, hi


def _logit_body(gl_ref, gr_ref, ea_ref, We_ref, be_ref, Wed_ref, attbd_ref,
                w_ref):
    ea = jnp.maximum(ea_ref[...] @ We_ref[...] + be_ref[...], 0.0)
    ep = ea @ Wed_ref[...]
    gla, glb = _unpack_bf16_pair(gl_ref[...])
    gra, grb = _unpack_bf16_pair(gr_ref[...])
    HALF = gla.shape[1]
    attbd = attbd_ref[...]
    m_lo = gla + gra + ep[:, :HALF]
    m_hi = glb + grb + ep[:, HALF:]
    s_lo = jnp.where(m_lo >= 0, m_lo, 0.2 * m_lo)
    s_hi = jnp.where(m_hi >= 0, m_hi, 0.2 * m_hi)
    expl = jnp.exp(s_lo @ attbd[:HALF] + s_hi @ attbd[HALF:])
    H = expl.shape[1]
    HID = 2 * HALF // H
    gl_heads = ([gla[:, h * HID:(h + 1) * HID] for h in range(H // 2)]
                + [glb[:, h * HID:(h + 1) * HID] for h in range(H // 2)])
    w_ref[...] = jnp.concatenate(
        [expl[:, h:h + 1] * gl_heads[h] for h in range(H)]
        + [expl, jnp.zeros((gla.shape[0], _WPAD), jnp.float32)],
        axis=1)


def _edge_weights(gl, gr, edge_attr, We, be, Wed, att_bd):
    E = gl.shape[0]
    HH = att_bd.shape[0]
    FE = edge_attr.shape[1]
    H = att_bd.shape[1]
    BE = 2000
    grid = (E // BE,)
    return pl.pallas_call(
        _logit_body,
        grid=grid,
        in_specs=[
            pl.BlockSpec((BE, HH // 2), lambda i: (i, 0)),
            pl.BlockSpec((BE, HH // 2), lambda i: (i, 0)),
            pl.BlockSpec((BE, FE), lambda i: (i, 0)),
            pl.BlockSpec(We.shape, lambda i: (0, 0)),
            pl.BlockSpec((1, be.shape[1]), lambda i: (0, 0)),
            pl.BlockSpec(Wed.shape, lambda i: (0, 0)),
            pl.BlockSpec(att_bd.shape, lambda i: (0, 0)),
        ],
        out_specs=pl.BlockSpec((BE, HH + H + _WPAD), lambda i: (i, 0)),
        out_shape=jax.ShapeDtypeStruct((E, HH + H + _WPAD), jnp.float32),
    )(gl, gr, edge_attr, We, be, Wed, att_bd)


def _dec_body(aggw_ref, bc_ref, Wd1_ref, bd1_ref, Wd2_ref, bd2_ref, out_ref):
    aggw = aggw_ref[...]
    HH = Wd1_ref.shape[0]
    H = aggw.shape[1] - HH - _WPAD
    HID = HH // H
    den = aggw[:, HH:HH + H]
    c = jnp.concatenate(
        [aggw[:, h * HID:(h + 1) * HID] / (den[:, h:h + 1] + 1e-16)
         for h in range(H)], axis=1) + bc_ref[...]
    d = jnp.maximum(c @ Wd1_ref[...] + bd1_ref[...], 0.0)
    out_ref[...] = d @ Wd2_ref[...] + bd2_ref[...]


def _decode(aggw, bias_conv, Wd1, bd1, Wd2, bd2):
    N, W = aggw.shape
    HH = Wd1.shape[0]
    OUT = Wd2.shape[1]
    BN = 2000
    grid = (N // BN,)
    return pl.pallas_call(
        _dec_body,
        grid=grid,
        in_specs=[
            pl.BlockSpec((BN, W), lambda i: (i, 0)),
            pl.BlockSpec((1, HH), lambda i: (0, 0)),
            pl.BlockSpec(Wd1.shape, lambda i: (0, 0)),
            pl.BlockSpec((1, bd1.shape[1]), lambda i: (0, 0)),
            pl.BlockSpec(Wd2.shape, lambda i: (0, 0)),
            pl.BlockSpec((1, bd2.shape[1]), lambda i: (0, 0)),
        ],
        out_specs=pl.BlockSpec((BN, OUT), lambda i: (i, 0)),
        out_shape=jax.ShapeDtypeStruct((N, OUT), jnp.float32),
    )(aggw, bias_conv, Wd1, bd1, Wd2, bd2)


def kernel(x, edge_index, edge_attr, Wn, bn, We, be, Wl, bl, Wr, br, Wed,
           att, bias_conv, Wd1, bd1, Wd2, bd2):
    N = x.shape[0]
    E = edge_index.shape[1]
    HEADS, HID = att.shape
    HH = HEADS * HID

    src = edge_index[0]
    dst = edge_index[1]

    att_bd = jnp.zeros((HH, HEADS), jnp.float32)
    for h in range(HEADS):
        att_bd = att_bd.at[h * HID:(h + 1) * HID, h].set(att[h])

    xl, xr = _node_encode(x, Wn, bn.reshape(1, -1), Wl, bl.reshape(1, -1),
                          Wr, br.reshape(1, -1))

    gl, gr = _sc_gather_pair(xl, xr, src, dst)

    w = _edge_weights(gl, gr, edge_attr, We, be.reshape(1, -1), Wed, att_bd)

    aggw = jax.ops.segment_sum(w, dst, num_segments=N)


    return _decode(aggw, bias_conv.reshape(1, -1), Wd1,
                   bd1.reshape(1, -1), Wd2, bd2.reshape(1, -1))


# speedup vs baseline: 1.3627x; 1.0003x over previous
import functools

import jax
import jax.numpy as jnp
from jax import lax
from jax.experimental import pallas as pl
from jax.experimental.pallas import tpu as pltpu
from jax.experimental.pallas import tpu_sc as plsc

_NC = 2
_NS = 16
_NW = _NC * _NS

_WPAD = 12


def _enc_body(x_ref, Wn_ref, bn_ref, Wl_ref, bl_ref, Wr_ref, br_ref,
              xl_ref, xr_ref):
    h = jnp.maximum(x_ref[...] @ Wn_ref[...] + bn_ref[...], 0.0)

    def pack(v):
        half = v.shape[1] // 2
        lo = v[:, :half].astype(jnp.bfloat16).astype(jnp.float32)
        hi = v[:, half:].astype(jnp.bfloat16).astype(jnp.float32)
        lo_u = jax.lax.bitcast_convert_type(lo, jnp.uint32) >> 16
        hi_u = jax.lax.bitcast_convert_type(hi, jnp.uint32) & jnp.uint32(
            0xFFFF0000)
        return lo_u | hi_u

    xl_ref[...] = pack(h @ Wl_ref[...] + bl_ref[...])
    xr_ref[...] = pack(h @ Wr_ref[...] + br_ref[...])


def _node_encode(x, Wn, bn, Wl, bl, Wr, br):
    N, F = x.shape
    HH = Wl.shape[1]
    BN = 2000
    grid = (N // BN,)
    return pl.pallas_call(
        _enc_body,
        grid=grid,
        in_specs=[
            pl.BlockSpec((BN, F), lambda i: (i, 0)),
            pl.BlockSpec(Wn.shape, lambda i: (0, 0)),
            pl.BlockSpec((1, bn.shape[1]), lambda i: (0, 0)),
            pl.BlockSpec(Wl.shape, lambda i: (0, 0)),
            pl.BlockSpec((1, bl.shape[1]), lambda i: (0, 0)),
            pl.BlockSpec(Wr.shape, lambda i: (0, 0)),
            pl.BlockSpec((1, br.shape[1]), lambda i: (0, 0)),
        ],
        out_specs=[
            pl.BlockSpec((BN, HH // 2), lambda i: (i, 0)),
            pl.BlockSpec((BN, HH // 2), lambda i: (i, 0)),
        ],
        out_shape=[
            jax.ShapeDtypeStruct((N, HH // 2), jnp.uint32),
            jax.ShapeDtypeStruct((N, HH // 2), jnp.uint32),
        ],
    )(x, Wn, bn, Wl, bl, Wr, br)


def _sc_gather_pair(xl, xr, src, dst):
    N, D = xl.shape
    E = src.shape[0]
    per_w = E // _NW
    C = next(c for c in (80, 40, 16, 8) if per_w % c == 0)
    steps = per_w // C
    main_steps = steps - (steps % 2)

    mesh = plsc.VectorSubcoreMesh(core_axis_name="c", subcore_axis_name="s")

    @functools.partial(
        pl.kernel,
        mesh=mesh,
        out_type=[
            jax.ShapeDtypeStruct((E, D), xl.dtype),
            jax.ShapeDtypeStruct((E, D), xr.dtype),
        ],
        scratch_types=[
            pltpu.VMEM((per_w,), jnp.int32),
            pltpu.VMEM((2, C, D), xl.dtype),
            pltpu.SemaphoreType.DMA((2,)),
            pltpu.SemaphoreType.DMA((2,)),
            pltpu.SemaphoreType.DMA,
        ],
    )
    def k(xl_hbm, xr_hbm, src_hbm, dst_hbm, gl_hbm, gr_hbm,
          idx_v, rows_v, gsem, osem, isem):
        wid = lax.axis_index("s") * _NC + lax.axis_index("c")
        base = wid * per_w

        def one_table(table_hbm, idx_hbm, out_hbm):
            pltpu.async_copy(idx_hbm.at[pl.ds(base, per_w)], idx_v, isem).wait()

            def gather(g, b):
                return pltpu.make_async_copy(
                    table_hbm.at[idx_v.at[pl.ds(g * C, C)]],
                    rows_v.at[b], gsem.at[b])

            def out_copy(g, b):
                return pltpu.make_async_copy(
                    rows_v.at[b], out_hbm.at[pl.ds(base + g * C, C)],
                    osem.at[b])

            gather(0, 0).start()

            @pl.loop(0, main_steps, step=2)
            def _(g0):
                for b in range(2):
                    g = g0 + b
                    nb = 1 - b
                    gather(g, b).wait()

                    @pl.when(g + 1 < steps)
                    def _():
                        @pl.when(g >= 1)
                        def _():
                            out_copy(g - 1, nb).wait()
                        gather(g + 1, nb).start()

                    out_copy(g, b).start()

            if steps % 2 == 1:
                gather(steps - 1, (steps - 1) % 2).wait()
                out_copy(steps - 1, (steps - 1) % 2).start()
            out_copy(steps - 2, (steps - 2) % 2).wait()
            out_copy(steps - 1, (steps - 1) % 2).wait()

        one_table(xl_hbm, src_hbm, gl_hbm)
        one_table(xr_hbm, dst_hbm, gr_hbm)

    return k(xl, xr, src, dst)


def _unpack_bf16_pair(p):
    lo = jax.lax.bitcast_convert_type(p << 16, jnp.float32)
    hi = jax.lax.bitcast_convert_type(p & jnp.uint32(0xFFFF0000), jnp.float32)
    return lo, hi


def _logit_body(gl_ref, gr_ref, ea_ref, We_ref, be_ref, Wed_ref, attbd_ref,
                w_ref):
    ea = jnp.maximum(ea_ref[...] @ We_ref[...] + be_ref[...], 0.0)
    ep = ea @ Wed_ref[...]
    gla, glb = _unpack_bf16_pair(gl_ref[...])
    gra, grb = _unpack_bf16_pair(gr_ref[...])
    HALF = gla.shape[1]
    attbd = attbd_ref[...]
    m_lo = gla + gra + ep[:, :HALF]
    m_hi = glb + grb + ep[:, HALF:]
    s_lo = jnp.where(m_lo >= 0, m_lo, 0.2 * m_lo)
    s_hi = jnp.where(m_hi >= 0, m_hi, 0.2 * m_hi)
    expl = jnp.exp(s_lo @ attbd[:HALF] + s_hi @ attbd[HALF:])
    H = expl.shape[1]
    HID = 2 * HALF // H
    gl_heads = ([gla[:, h * HID:(h + 1) * HID] for h in range(H // 2)]
                + [glb[:, h * HID:(h + 1) * HID] for h in range(H // 2)])
    w_ref[...] = jnp.concatenate(
        [expl[:, h:h + 1] * gl_heads[h] for h in range(H)]
        + [expl, jnp.zeros((gla.shape[0], _WPAD), jnp.float32)],
        axis=1)


def _edge_weights(gl, gr, edge_attr, We, be, Wed, att_bd):
    E = gl.shape[0]
    HH = att_bd.shape[0]
    FE = edge_attr.shape[1]
    H = att_bd.shape[1]
    BE = 2000
    grid = (E // BE,)
    return pl.pallas_call(
        _logit_body,
        grid=grid,
        in_specs=[
            pl.BlockSpec((BE, HH // 2), lambda i: (i, 0)),
            pl.BlockSpec((BE, HH // 2), lambda i: (i, 0)),
            pl.BlockSpec((BE, FE), lambda i: (i, 0)),
            pl.BlockSpec(We.shape, lambda i: (0, 0)),
            pl.BlockSpec((1, be.shape[1]), lambda i: (0, 0)),
            pl.BlockSpec(Wed.shape, lambda i: (0, 0)),
            pl.BlockSpec(att_bd.shape, lambda i: (0, 0)),
        ],
        out_specs=pl.BlockSpec((BE, HH + H + _WPAD), lambda i: (i, 0)),
        out_shape=jax.ShapeDtypeStruct((E, HH + H + _WPAD), jnp.float32),
    )(gl, gr, edge_attr, We, be, Wed, att_bd)


def _dec_body(aggw_ref, bc_ref, Wd1_ref, bd1_ref, Wd2_ref, bd2_ref, out_ref):
    aggw = aggw_ref[...]
    HH = Wd1_ref.shape[0]
    H = aggw.shape[1] - HH - _WPAD
    HID = HH // H
    den = aggw[:, HH:HH + H]
    c = jnp.concatenate(
        [aggw[:, h * HID:(h + 1) * HID] / (den[:, h:h + 1] + 1e-16)
         for h in range(H)], axis=1) + bc_ref[...]
    d = jnp.maximum(c @ Wd1_ref[...] + bd1_ref[...], 0.0)
    out_ref[...] = d @ Wd2_ref[...] + bd2_ref[...]


def _decode(aggw, bias_conv, Wd1, bd1, Wd2, bd2):
    N, W = aggw.shape
    HH = Wd1.shape[0]
    OUT = Wd2.shape[1]
    BN = 2000
    grid = (N // BN,)
    return pl.pallas_call(
        _dec_body,
        grid=grid,
        in_specs=[
            pl.BlockSpec((BN, W), lambda i: (i, 0)),
            pl.BlockSpec((1, HH), lambda i: (0, 0)),
            pl.BlockSpec(Wd1.shape, lambda i: (0, 0)),
            pl.BlockSpec((1, bd1.shape[1]), lambda i: (0, 0)),
            pl.BlockSpec(Wd2.shape, lambda i: (0, 0)),
            pl.BlockSpec((1, bd2.shape[1]), lambda i: (0, 0)),
        ],
        out_specs=pl.BlockSpec((BN, OUT), lambda i: (i, 0)),
        out_shape=jax.ShapeDtypeStruct((N, OUT), jnp.float32),
    )(aggw, bias_conv, Wd1, bd1, Wd2, bd2)


def kernel(x, edge_index, edge_attr, Wn, bn, We, be, Wl, bl, Wr, br, Wed,
           att, bias_conv, Wd1, bd1, Wd2, bd2):
    N = x.shape[0]
    E = edge_index.shape[1]
    HEADS, HID = att.shape
    HH = HEADS * HID

    src = edge_index[0]
    dst = edge_index[1]

    att_bd = jnp.zeros((HH, HEADS), jnp.float32)
    for h in range(HEADS):
        att_bd = att_bd.at[h * HID:(h + 1) * HID, h].set(att[h])

    xl, xr = _node_encode(x, Wn, bn.reshape(1, -1), Wl, bl.reshape(1, -1),
                          Wr, br.reshape(1, -1))

    gl, gr = _sc_gather_pair(xl, xr, src, dst)

    w = _edge_weights(gl, gr, edge_attr, We, be.reshape(1, -1), Wed, att_bd)

    aggw = jax.ops.segment_sum(w, dst, num_segments=N)

    return _decode(aggw, bias_conv.reshape(1, -1), Wd1,
                   bd1.reshape(1, -1), Wd2, bd2.reshape(1, -1))
